# baseline (speedup 1.0000x reference)
import functools

import jax
import jax.numpy as jnp
from jax import lax
from jax.experimental import pallas as pl
from jax.experimental.pallas import tpu as pltpu
from jax.experimental.pallas import tpu_sc as plsc

_N = 10000
_NP = 10240
_E = 320000
_CH = 128
_CPT = 80
_EPT = _CH * _CPT
_E_PAD = 32 * _EPT
_ROWS_PER_TILE = _NP // 16

_MESH = plsc.VectorSubcoreMesh(core_axis_name="c", subcore_axis_name="s")


@functools.partial(
    pl.kernel,
    out_type=jax.ShapeDtypeStruct((2 * _NP,), jnp.float32),
    mesh=_MESH,
    scratch_types=[
        pltpu.VMEM_SHARED((_NP,), jnp.float32),
        pltpu.VMEM((_CPT, _CH), jnp.int32),
        pltpu.VMEM((_CPT, _CH), jnp.float32),
        pltpu.VMEM((_ROWS_PER_TILE,), jnp.float32),
    ],
)
def _sc_deg(row3d_hbm, ew3d_hbm, out_hbm, acc_sh, rowv, ewv, zb):
    c = lax.axis_index("c")
    s = lax.axis_index("s")
    wid = c * 16 + s

    def _zero(i, carry):
        zb[pl.ds(i * 16, 16)] = jnp.zeros((16,), jnp.float32)
        return carry

    lax.fori_loop(0, _ROWS_PER_TILE // 16, _zero, 0)
    pltpu.sync_copy(zb, acc_sh.at[pl.ds(s * _ROWS_PER_TILE, _ROWS_PER_TILE)])
    pltpu.sync_copy(row3d_hbm.at[wid], rowv)
    pltpu.sync_copy(ew3d_hbm.at[wid], ewv)
    plsc.subcore_barrier()

    def _chunk(i, carry):
        pltpu.sync_copy(ewv.at[i], acc_sh.at[rowv.at[i]], add=True)
        return carry

    lax.fori_loop(0, _CPT, _chunk, 0)
    plsc.subcore_barrier()
    pltpu.sync_copy(
        acc_sh.at[pl.ds(s * _ROWS_PER_TILE, _ROWS_PER_TILE)],
        out_hbm.at[pl.ds(c * _NP + s * _ROWS_PER_TILE, _ROWS_PER_TILE)])


@functools.partial(
    pl.kernel,
    out_type=jax.ShapeDtypeStruct((2, _NP, 16), jnp.float32),
    mesh=_MESH,
    scratch_types=[
        pltpu.VMEM_SHARED((_NP, 16), jnp.float32),
        pltpu.VMEM((_CPT, _CH), jnp.int32),
        pltpu.VMEM((_CPT, _CH), jnp.int32),
        pltpu.VMEM((_EPT,), jnp.float32),
        pltpu.VMEM((4, _CH, 16), jnp.float32),
        pltpu.VMEM((4, _CH, 16), jnp.float32),
        pltpu.VMEM((_ROWS_PER_TILE, 16), jnp.float32),
        [pltpu.SemaphoreType.DMA] * 4,
        [pltpu.SemaphoreType.DMA] * 4,
    ],
    compiler_params=pltpu.CompilerParams(use_tc_tiling_on_sc=False),
)
def _sc_spmm(col3d_hbm, row3d_hbm, ew1d_hbm, y_hbm, out_hbm,
             acc_sh, colv, rowv, ewv, gb, sb, zb, gsem, ssem):
    c = lax.axis_index("c")
    s = lax.axis_index("s")
    wid = c * 16 + s

    def _zero(i, carry):
        zb[i] = jnp.zeros((16,), jnp.float32)
        return carry

    lax.fori_loop(0, _ROWS_PER_TILE, _zero, 0)
    pltpu.sync_copy(zb, acc_sh.at[pl.ds(s * _ROWS_PER_TILE, _ROWS_PER_TILE)])
    pltpu.sync_copy(col3d_hbm.at[wid], colv)
    pltpu.sync_copy(row3d_hbm.at[wid], rowv)
    pltpu.sync_copy(ew1d_hbm.at[pl.ds(wid * _EPT, _EPT)], ewv)
    plsc.subcore_barrier()

    def _fire_gather(ci, k):
        pltpu.async_copy(y_hbm.at[colv.at[ci]], gb.at[k], gsem[k])

    def _wait_gather(ci, k):
        pltpu.make_async_copy(y_hbm.at[colv.at[ci]], gb.at[k], gsem[k]).wait()

    def _fire_scatter(ci, k):
        pltpu.async_copy(sb.at[k], acc_sh.at[rowv.at[ci]], ssem[k], add=True)

    def _wait_scatter(ci, k):
        pltpu.make_async_copy(sb.at[k], acc_sh.at[rowv.at[ci]],
                              ssem[k]).wait()

    def _scale(ci, k):
        def _grp(g, carry):
            wv = ewv[pl.ds(ci * _CH + g * 16, 16)]
            e0 = g * 16
            for j in range(16):
                sb[k, e0 + j] = gb[k, e0 + j] * wv[j]
            return carry

        lax.fori_loop(0, _CH // 16, _grp, 0, unroll=2)

    for k in range(4):
        _fire_gather(k, k)
    for k in range(4):
        _wait_gather(k, k)
        _scale(k, k)
        _fire_gather(k + 4, k)
        _fire_scatter(k, k)

    def _body(g, carry):
        for k in range(4):
            ci = g * 4 + k
            _wait_gather(ci, k)
            _wait_scatter(ci - 4, k)
            _scale(ci, k)
            _fire_gather(ci + 4, k)
            _fire_scatter(ci, k)
        return carry

    lax.fori_loop(1, _CPT // 4 - 1, _body, 0)

    for k in range(4):
        ci = _CPT - 4 + k
        _wait_gather(ci, k)
        _wait_scatter(ci - 4, k)
        _scale(ci, k)
        _fire_scatter(ci, k)
    for k in range(4):
        _wait_scatter(_CPT - 4 + k, k)

    plsc.subcore_barrier()
    pltpu.sync_copy(acc_sh.at[pl.ds(s * _ROWS_PER_TILE, _ROWS_PER_TILE)],
                    out_hbm.at[c, pl.ds(s * _ROWS_PER_TILE, _ROWS_PER_TILE)])


def _tc0_body(x_ref, w_ref, o_ref):
    o_ref[...] = jnp.dot(x_ref[...], w_ref[...],
                         preferred_element_type=jnp.float32)


def _tc1b_body(o_ref, deg_ref, xw0_ref, y1p_ref, dis_ref):
    d2 = deg_ref[...]
    d = d2[:, 0:1] + d2[:, 1:2]
    dis = jnp.where(d > 0, lax.rsqrt(jnp.maximum(d, 1e-30)), 0.0)
    dis_ref[...] = dis
    o = o_ref[...]
    xw0_ref[...] = o[:, :16]
    y1p_ref[pl.ds(0, _N), :] = dis * o[:, 16:]
    y1p_ref[pl.ds(_N, _NP - _N), :] = jnp.zeros((_NP - _N, 16), jnp.float32)


def _tc2_body(xw0_ref, s1p_ref, dis_ref, b1_ref, w_ref,
              hw0_ref, y2p_ref):
    dis = dis_ref[...]
    s1 = s1p_ref[0, pl.ds(0, _N), :] + s1p_ref[1, pl.ds(0, _N), :]
    h = jnp.maximum(xw0_ref[...] - dis * s1 + b1_ref[...], 0.0)
    o = jnp.dot(h, w_ref[...], preferred_element_type=jnp.float32)
    hw0_ref[...] = o[:, :16]
    y2p_ref[pl.ds(0, _N), :] = dis * o[:, 16:]
    y2p_ref[pl.ds(_N, _NP - _N), :] = jnp.zeros((_NP - _N, 16), jnp.float32)


def _tc3_body(hw0_ref, s2p_ref, dis_ref, b2_ref, out_ref):
    s2 = s2p_ref[0, pl.ds(0, _N), :] + s2p_ref[1, pl.ds(0, _N), :]
    o = hw0_ref[...] - dis_ref[...] * s2 + b2_ref[...]
    m = jnp.max(o, axis=1, keepdims=True)
    z = o - m
    lse = jnp.log(jnp.sum(jnp.exp(z), axis=1, keepdims=True))
    out_ref[...] = z - lse


_f32 = jnp.float32


def kernel(x, edge_index, edge_weight, W1, b1, W2, b2):
    row = edge_index[0]
    col = edge_index[1]
    pad = _E_PAD - _E
    row_p = jnp.concatenate([row, jnp.zeros((pad,), jnp.int32)])
    col_p = jnp.concatenate([col, jnp.zeros((pad,), jnp.int32)])
    ew_p = jnp.concatenate([edge_weight, jnp.zeros((pad,), _f32)])
    row3d = row_p.reshape(32, _CPT, _CH)
    col3d = col_p.reshape(32, _CPT, _CH)
    ew3d = ew_p.reshape(32, _CPT, _CH)

    deg2 = _sc_deg(row3d, ew3d).reshape(2, _NP)
    degT = deg2[:, :_N].T

    w1cat = jnp.concatenate([W1[0], W1[1]], axis=1)
    w2cat = jnp.concatenate([W2[0], W2[1]], axis=1)

    o1 = pl.pallas_call(
        _tc0_body,
        out_shape=jax.ShapeDtypeStruct((_N, 32), _f32),
    )(x, w1cat)

    xw0, y1p_pad, dis = pl.pallas_call(
        _tc1b_body,
        out_shape=[
            jax.ShapeDtypeStruct((_N, 16), _f32),
            jax.ShapeDtypeStruct((_NP, 16), _f32),
            jax.ShapeDtypeStruct((_N, 1), _f32),
        ],
    )(o1, degT)

    s1p = _sc_spmm(col3d, row3d, ew_p, y1p_pad)

    hw0, y2p_pad = pl.pallas_call(
        _tc2_body,
        out_shape=[
            jax.ShapeDtypeStruct((_N, 16), _f32),
            jax.ShapeDtypeStruct((_NP, 16), _f32),
        ],
    )(xw0, s1p, dis, b1.reshape(1, 16), w2cat)

    s2p = _sc_spmm(col3d, row3d, ew_p, y2p_pad)

    out = pl.pallas_call(
        _tc3_body,
        out_shape=jax.ShapeDtypeStruct((_N, 16), _f32),
    )(hw0, s2p, dis, b2.reshape(1, 16))
    return out


# speedup vs baseline: 1.0554x; 1.0554x over previous
import functools

import jax
import jax.numpy as jnp
from jax import lax
from jax.experimental import pallas as pl
from jax.experimental.pallas import tpu as pltpu
from jax.experimental.pallas import tpu_sc as plsc

_N = 10000
_NP = 10240
_E = 320000
_CH = 128
_CPT = 80
_EPT = _CH * _CPT
_E_PAD = 32 * _EPT
_ROWS_PER_TILE = _NP // 16

_MESH = plsc.VectorSubcoreMesh(core_axis_name="c", subcore_axis_name="s")


@functools.partial(
    pl.kernel,
    out_type=jax.ShapeDtypeStruct((2 * _NP,), jnp.float32),
    mesh=_MESH,
    scratch_types=[
        pltpu.VMEM_SHARED((_NP,), jnp.float32),
        pltpu.VMEM((_CPT, _CH), jnp.int32),
        pltpu.VMEM((_CPT, _CH), jnp.float32),
        pltpu.VMEM((_ROWS_PER_TILE,), jnp.float32),
    ],
)
def _sc_deg(row3d_hbm, ew3d_hbm, out_hbm, acc_sh, rowv, ewv, zb):
    c = lax.axis_index("c")
    s = lax.axis_index("s")
    wid = c * 16 + s

    def _zero(i, carry):
        zb[pl.ds(i * 16, 16)] = jnp.zeros((16,), jnp.float32)
        return carry

    lax.fori_loop(0, _ROWS_PER_TILE // 16, _zero, 0)
    pltpu.sync_copy(zb, acc_sh.at[pl.ds(s * _ROWS_PER_TILE, _ROWS_PER_TILE)])
    pltpu.sync_copy(row3d_hbm.at[wid], rowv)
    pltpu.sync_copy(ew3d_hbm.at[wid], ewv)
    plsc.subcore_barrier()

    def _chunk(i, carry):
        pltpu.sync_copy(ewv.at[i], acc_sh.at[rowv.at[i]], add=True)
        return carry

    lax.fori_loop(0, _CPT, _chunk, 0)
    plsc.subcore_barrier()
    pltpu.sync_copy(
        acc_sh.at[pl.ds(s * _ROWS_PER_TILE, _ROWS_PER_TILE)],
        out_hbm.at[pl.ds(c * _NP + s * _ROWS_PER_TILE, _ROWS_PER_TILE)])


@functools.partial(
    pl.kernel,
    out_type=jax.ShapeDtypeStruct((2, _NP, 16), jnp.float32),
    mesh=_MESH,
    scratch_types=[
        pltpu.VMEM_SHARED((_NP, 16), jnp.float32),
        pltpu.VMEM((_CPT, _CH), jnp.int32),
        pltpu.VMEM((_CPT, _CH), jnp.int32),
        pltpu.VMEM((_EPT,), jnp.float32),
        pltpu.VMEM((4, _CH, 16), jnp.float32),
        pltpu.VMEM((4, _CH, 16), jnp.float32),
        pltpu.VMEM((_ROWS_PER_TILE, 16), jnp.float32),
        [pltpu.SemaphoreType.DMA] * 4,
        [pltpu.SemaphoreType.DMA] * 4,
    ],
    compiler_params=pltpu.CompilerParams(use_tc_tiling_on_sc=False),
)
def _sc_spmm(col3d_hbm, row3d_hbm, ew1d_hbm, y_hbm, out_hbm,
             acc_sh, colv, rowv, ewv, gb, sb, zb, gsem, ssem):
    c = lax.axis_index("c")
    s = lax.axis_index("s")
    wid = c * 16 + s

    def _zero(i, carry):
        zb[i] = jnp.zeros((16,), jnp.float32)
        return carry

    lax.fori_loop(0, _ROWS_PER_TILE, _zero, 0)
    pltpu.sync_copy(zb, acc_sh.at[pl.ds(s * _ROWS_PER_TILE, _ROWS_PER_TILE)])
    pltpu.sync_copy(col3d_hbm.at[wid], colv)
    pltpu.sync_copy(row3d_hbm.at[wid], rowv)
    pltpu.sync_copy(ew1d_hbm.at[pl.ds(wid * _EPT, _EPT)], ewv)
    plsc.subcore_barrier()

    def _fire_gather(ci, k):
        pltpu.async_copy(y_hbm.at[colv.at[ci]], gb.at[k], gsem[k])

    def _wait_gather(ci, k):
        pltpu.make_async_copy(y_hbm.at[colv.at[ci]], gb.at[k], gsem[k]).wait()

    def _fire_scatter(ci, k):
        pltpu.async_copy(sb.at[k], acc_sh.at[rowv.at[ci]], ssem[k], add=True)

    def _wait_scatter(ci, k):
        pltpu.make_async_copy(sb.at[k], acc_sh.at[rowv.at[ci]],
                              ssem[k]).wait()

    def _scale(ci, k):
        def _grp(g, carry):
            wv = ewv[pl.ds(ci * _CH + g * 16, 16)]
            e0 = g * 16
            for j in range(16):
                sb[k, e0 + j] = gb[k, e0 + j] * wv[j]
            return carry

        lax.fori_loop(0, _CH // 16, _grp, 0, unroll=2)

    for k in range(4):
        _fire_gather(k, k)
    for k in range(4):
        _wait_gather(k, k)
        _scale(k, k)
        _fire_gather(k + 4, k)
        _fire_scatter(k, k)

    def _body(g, carry):
        for k in range(4):
            ci = g * 4 + k
            _wait_gather(ci, k)
            _wait_scatter(ci - 4, k)
            _scale(ci, k)
            _fire_gather(ci + 4, k)
            _fire_scatter(ci, k)
        return carry

    lax.fori_loop(1, _CPT // 4 - 1, _body, 0)

    for k in range(4):
        ci = _CPT - 4 + k
        _wait_gather(ci, k)
        _wait_scatter(ci - 4, k)
        _scale(ci, k)
        _fire_scatter(ci, k)
    for k in range(4):
        _wait_scatter(_CPT - 4 + k, k)

    plsc.subcore_barrier()
    pltpu.sync_copy(acc_sh.at[pl.ds(s * _ROWS_PER_TILE, _ROWS_PER_TILE)],
                    out_hbm.at[c, pl.ds(s * _ROWS_PER_TILE, _ROWS_PER_TILE)])


def _tc1_body(x_ref, w_ref, deg_ref, xw0_ref, y1p_ref, dis_ref):
    d2 = deg_ref[...]
    d = d2[:, 0:1] + d2[:, 1:2]
    dis = jnp.where(d > 0, lax.rsqrt(jnp.maximum(d, 1e-30)), 0.0)
    dis_ref[...] = dis
    o = jnp.dot(x_ref[...], w_ref[...], preferred_element_type=jnp.float32)
    xw0_ref[...] = o[:, :16]
    y1p_ref[pl.ds(0, _N), :] = dis * o[:, 16:]
    y1p_ref[pl.ds(_N, _NP - _N), :] = jnp.zeros((_NP - _N, 16), jnp.float32)


def _tc2_body(xw0_ref, s1p_ref, dis_ref, b1_ref, w_ref,
              hw0_ref, y2p_ref):
    dis = dis_ref[...]
    s1 = s1p_ref[0, pl.ds(0, _N), :] + s1p_ref[1, pl.ds(0, _N), :]
    h = jnp.maximum(xw0_ref[...] - dis * s1 + b1_ref[...], 0.0)
    o = jnp.dot(h, w_ref[...], preferred_element_type=jnp.float32)
    hw0_ref[...] = o[:, :16]
    y2p_ref[pl.ds(0, _N), :] = dis * o[:, 16:]
    y2p_ref[pl.ds(_N, _NP - _N), :] = jnp.zeros((_NP - _N, 16), jnp.float32)


def _tc3_body(hw0_ref, s2p_ref, dis_ref, b2_ref, out_ref):
    s2 = s2p_ref[0, pl.ds(0, _N), :] + s2p_ref[1, pl.ds(0, _N), :]
    o = hw0_ref[...] - dis_ref[...] * s2 + b2_ref[...]
    m = jnp.max(o, axis=1, keepdims=True)
    z = o - m
    lse = jnp.log(jnp.sum(jnp.exp(z), axis=1, keepdims=True))
    out_ref[...] = z - lse


_f32 = jnp.float32


def kernel(x, edge_index, edge_weight, W1, b1, W2, b2):
    row = edge_index[0]
    col = edge_index[1]
    pad = _E_PAD - _E
    row_p = jnp.concatenate([row, jnp.zeros((pad,), jnp.int32)])
    col_p = jnp.concatenate([col, jnp.zeros((pad,), jnp.int32)])
    ew_p = jnp.concatenate([edge_weight, jnp.zeros((pad,), _f32)])
    row3d = row_p.reshape(32, _CPT, _CH)
    col3d = col_p.reshape(32, _CPT, _CH)
    ew3d = ew_p.reshape(32, _CPT, _CH)

    deg2 = _sc_deg(row3d, ew3d).reshape(2, _NP)
    degT = deg2[:, :_N].T

    w1cat = jnp.concatenate([W1[0], W1[1]], axis=1)
    w2cat = jnp.concatenate([W2[0], W2[1]], axis=1)

    xw0, y1p_pad, dis = pl.pallas_call(
        _tc1_body,
        out_shape=[
            jax.ShapeDtypeStruct((_N, 16), _f32),
            jax.ShapeDtypeStruct((_NP, 16), _f32),
            jax.ShapeDtypeStruct((_N, 1), _f32),
        ],
    )(x, w1cat, degT)

    s1p = _sc_spmm(col3d, row3d, ew_p, y1p_pad)

    hw0, y2p_pad = pl.pallas_call(
        _tc2_body,
        out_shape=[
            jax.ShapeDtypeStruct((_N, 16), _f32),
            jax.ShapeDtypeStruct((_NP, 16), _f32),
        ],
    )(xw0, s1p, dis, b1.reshape(1, 16), w2cat)

    s2p = _sc_spmm(col3d, row3d, ew_p, y2p_pad)

    out = pl.pallas_call(
        _tc3_body,
        out_shape=jax.ShapeDtypeStruct((_N, 16), _f32),
    )(hw0, s2p, dis, b2.reshape(1, 16))
    return out


# speedup vs baseline: 1.1426x; 1.0826x over previous
import functools

import jax
import jax.numpy as jnp
from jax import lax
from jax.experimental import pallas as pl
from jax.experimental.pallas import tpu as pltpu
from jax.experimental.pallas import tpu_sc as plsc

_N = 10000
_NP = 10240
_E = 320000
_CH = 128
_CPT = 80
_EPT = _CH * _CPT
_E_PAD = 32 * _EPT
_ROWS_PER_TILE = _NP // 16

_MESH = plsc.VectorSubcoreMesh(core_axis_name="c", subcore_axis_name="s")


@functools.partial(
    pl.kernel,
    out_type=jax.ShapeDtypeStruct((2 * _NP,), jnp.float32),
    mesh=_MESH,
    scratch_types=[
        pltpu.VMEM_SHARED((_NP,), jnp.float32),
        pltpu.VMEM((_CPT, _CH), jnp.int32),
        pltpu.VMEM((_CPT, _CH), jnp.float32),
        pltpu.VMEM((_ROWS_PER_TILE,), jnp.float32),
    ],
)
def _sc_deg(row3d_hbm, ew3d_hbm, out_hbm, acc_sh, rowv, ewv, zb):
    c = lax.axis_index("c")
    s = lax.axis_index("s")
    wid = c * 16 + s

    def _zero(i, carry):
        zb[pl.ds(i * 16, 16)] = jnp.zeros((16,), jnp.float32)
        return carry

    lax.fori_loop(0, _ROWS_PER_TILE // 16, _zero, 0)
    pltpu.sync_copy(zb, acc_sh.at[pl.ds(s * _ROWS_PER_TILE, _ROWS_PER_TILE)])
    pltpu.sync_copy(row3d_hbm.at[wid], rowv)
    pltpu.sync_copy(ew3d_hbm.at[wid], ewv)
    plsc.subcore_barrier()

    def _chunk(i, carry):
        pltpu.sync_copy(ewv.at[i], acc_sh.at[rowv.at[i]], add=True)
        return carry

    lax.fori_loop(0, _CPT, _chunk, 0)
    plsc.subcore_barrier()
    pltpu.sync_copy(
        acc_sh.at[pl.ds(s * _ROWS_PER_TILE, _ROWS_PER_TILE)],
        out_hbm.at[pl.ds(c * _NP + s * _ROWS_PER_TILE, _ROWS_PER_TILE)])


@functools.partial(
    pl.kernel,
    out_type=jax.ShapeDtypeStruct((2, _NP, 16), jnp.float32),
    mesh=_MESH,
    scratch_types=[
        pltpu.VMEM_SHARED((_NP, 16), jnp.float32),
        pltpu.VMEM((_CPT, _CH), jnp.int32),
        pltpu.VMEM((_CPT, _CH), jnp.int32),
        pltpu.VMEM((_EPT,), jnp.float32),
        pltpu.VMEM((4, _CH, 16), jnp.float32),
        pltpu.VMEM((4, _CH, 16), jnp.float32),
        pltpu.VMEM((_ROWS_PER_TILE, 16), jnp.float32),
        [pltpu.SemaphoreType.DMA] * 4,
        [pltpu.SemaphoreType.DMA] * 4,
    ],
    compiler_params=pltpu.CompilerParams(use_tc_tiling_on_sc=False),
)
def _sc_spmm(col3d_hbm, row3d_hbm, ew1d_hbm, y_hbm, out_hbm,
             acc_sh, colv, rowv, ewv, gb, sb, zb, gsem, ssem):
    c = lax.axis_index("c")
    s = lax.axis_index("s")
    wid = c * 16 + s

    def _zero(i, carry):
        zb[i] = jnp.zeros((16,), jnp.float32)
        return carry

    lax.fori_loop(0, _ROWS_PER_TILE, _zero, 0)
    pltpu.sync_copy(zb, acc_sh.at[pl.ds(s * _ROWS_PER_TILE, _ROWS_PER_TILE)])
    pltpu.sync_copy(col3d_hbm.at[wid], colv)
    pltpu.sync_copy(row3d_hbm.at[wid], rowv)
    pltpu.sync_copy(ew1d_hbm.at[pl.ds(wid * _EPT, _EPT)], ewv)
    plsc.subcore_barrier()

    def _fire_gather(ci, k):
        pltpu.async_copy(y_hbm.at[colv.at[ci]], gb.at[k], gsem[k])

    def _wait_gather(ci, k):
        pltpu.make_async_copy(y_hbm.at[colv.at[ci]], gb.at[k], gsem[k]).wait()

    def _fire_scatter(ci, k):
        pltpu.async_copy(sb.at[k], acc_sh.at[rowv.at[ci]], ssem[k], add=True)

    def _wait_scatter(ci, k):
        pltpu.make_async_copy(sb.at[k], acc_sh.at[rowv.at[ci]],
                              ssem[k]).wait()

    def _scale(ci, k):
        def _grp(g, carry):
            wv = ewv[pl.ds(ci * _CH + g * 16, 16)]
            e0 = g * 16
            for j in range(16):
                sb[k, e0 + j] = gb[k, e0 + j] * wv[j]
            return carry

        lax.fori_loop(0, _CH // 16, _grp, 0)

    for k in range(4):
        _fire_gather(k, k)
    for k in range(4):
        _wait_gather(k, k)
        _scale(k, k)
        _fire_gather(k + 4, k)
        _fire_scatter(k, k)

    def _body(g, carry):
        for k in range(4):
            ci = g * 4 + k
            _wait_gather(ci, k)
            _wait_scatter(ci - 4, k)
            _scale(ci, k)
            _fire_gather(ci + 4, k)
            _fire_scatter(ci, k)
        return carry

    lax.fori_loop(1, _CPT // 4 - 1, _body, 0)

    for k in range(4):
        ci = _CPT - 4 + k
        _wait_gather(ci, k)
        _wait_scatter(ci - 4, k)
        _scale(ci, k)
        _fire_scatter(ci, k)
    for k in range(4):
        _wait_scatter(_CPT - 4 + k, k)

    plsc.subcore_barrier()
    pltpu.sync_copy(acc_sh.at[pl.ds(s * _ROWS_PER_TILE, _ROWS_PER_TILE)],
                    out_hbm.at[c, pl.ds(s * _ROWS_PER_TILE, _ROWS_PER_TILE)])


def _tc1_body(x_ref, w_ref, deg_ref, xw0_ref, y1p_ref, dis_ref):
    d2 = deg_ref[...]
    d = d2[:, 0:1] + d2[:, 1:2]
    dis = jnp.where(d > 0, lax.rsqrt(jnp.maximum(d, 1e-30)), 0.0)
    dis_ref[...] = dis
    o = jnp.dot(x_ref[...], w_ref[...], preferred_element_type=jnp.float32)
    xw0_ref[...] = o[:, :16]
    y1p_ref[pl.ds(0, _N), :] = dis * o[:, 16:]
    y1p_ref[pl.ds(_N, _NP - _N), :] = jnp.zeros((_NP - _N, 16), jnp.float32)


def _tc2_body(xw0_ref, s1p_ref, dis_ref, b1_ref, w_ref,
              hw0_ref, y2p_ref):
    dis = dis_ref[...]
    s1 = s1p_ref[0, pl.ds(0, _N), :] + s1p_ref[1, pl.ds(0, _N), :]
    h = jnp.maximum(xw0_ref[...] - dis * s1 + b1_ref[...], 0.0)
    o = jnp.dot(h, w_ref[...], preferred_element_type=jnp.float32)
    hw0_ref[...] = o[:, :16]
    y2p_ref[pl.ds(0, _N), :] = dis * o[:, 16:]
    y2p_ref[pl.ds(_N, _NP - _N), :] = jnp.zeros((_NP - _N, 16), jnp.float32)


def _tc3_body(hw0_ref, s2p_ref, dis_ref, b2_ref, out_ref):
    s2 = s2p_ref[0, pl.ds(0, _N), :] + s2p_ref[1, pl.ds(0, _N), :]
    o = hw0_ref[...] - dis_ref[...] * s2 + b2_ref[...]
    m = jnp.max(o, axis=1, keepdims=True)
    z = o - m
    lse = jnp.log(jnp.sum(jnp.exp(z), axis=1, keepdims=True))
    out_ref[...] = z - lse


_f32 = jnp.float32


def kernel(x, edge_index, edge_weight, W1, b1, W2, b2):
    row = edge_index[0]
    col = edge_index[1]
    pad = _E_PAD - _E
    row_p = jnp.concatenate([row, jnp.zeros((pad,), jnp.int32)])
    col_p = jnp.concatenate([col, jnp.zeros((pad,), jnp.int32)])
    ew_p = jnp.concatenate([edge_weight, jnp.zeros((pad,), _f32)])
    row3d = row_p.reshape(32, _CPT, _CH)
    col3d = col_p.reshape(32, _CPT, _CH)
    ew3d = ew_p.reshape(32, _CPT, _CH)

    deg2 = _sc_deg(row3d, ew3d).reshape(2, _NP)
    degT = deg2[:, :_N].T

    w1cat = jnp.concatenate([W1[0], W1[1]], axis=1)
    w2cat = jnp.concatenate([W2[0], W2[1]], axis=1)

    xw0, y1p_pad, dis = pl.pallas_call(
        _tc1_body,
        out_shape=[
            jax.ShapeDtypeStruct((_N, 16), _f32),
            jax.ShapeDtypeStruct((_NP, 16), _f32),
            jax.ShapeDtypeStruct((_N, 1), _f32),
        ],
    )(x, w1cat, degT)

    s1p = _sc_spmm(col3d, row3d, ew_p, y1p_pad)

    hw0, y2p_pad = pl.pallas_call(
        _tc2_body,
        out_shape=[
            jax.ShapeDtypeStruct((_N, 16), _f32),
            jax.ShapeDtypeStruct((_NP, 16), _f32),
        ],
    )(xw0, s1p, dis, b1.reshape(1, 16), w2cat)

    s2p = _sc_spmm(col3d, row3d, ew_p, y2p_pad)

    out = pl.pallas_call(
        _tc3_body,
        out_shape=jax.ShapeDtypeStruct((_N, 16), _f32),
    )(hw0, s2p, dis, b2.reshape(1, 16))
    return out


# speedup vs baseline: 1.2027x; 1.0527x over previous
import functools

import jax
import jax.numpy as jnp
from jax import lax
from jax.experimental import pallas as pl
from jax.experimental.pallas import tpu as pltpu
from jax.experimental.pallas import tpu_sc as plsc

_N = 10000
_NP = 10240
_PR = _NP // 8
_E = 320000
_CH = 128
_CPT = 80
_EPT = _CH * _CPT
_E_PAD = 32 * _EPT
_RPT = _NP // 16

_MESH = plsc.VectorSubcoreMesh(core_axis_name="c", subcore_axis_name="s")


@functools.partial(
    pl.kernel,
    out_type=jax.ShapeDtypeStruct((2, _NP, 16), jnp.float32),
    mesh=_MESH,
    scratch_types=[
        pltpu.VMEM_SHARED((_NP,), jnp.float32),
        pltpu.VMEM((_CPT, _CH), jnp.int32),
        pltpu.VMEM((_CPT, _CH), jnp.float32),
        pltpu.VMEM((_RPT,), jnp.float32),
        pltpu.VMEM((_RPT, 16), jnp.float32),
    ],
)
def _sc_deg(row3d_hbm, ew3d_hbm, out_hbm, acc_sh, rowv, ewv, zb, rep):
    c = lax.axis_index("c")
    s = lax.axis_index("s")
    wid = c * 16 + s

    def _zero(i, carry):
        zb[pl.ds(i * 16, 16)] = jnp.zeros((16,), jnp.float32)
        return carry

    lax.fori_loop(0, _RPT // 16, _zero, 0)
    pltpu.sync_copy(zb, acc_sh.at[pl.ds(s * _RPT, _RPT)])
    pltpu.sync_copy(row3d_hbm.at[wid], rowv)
    pltpu.sync_copy(ew3d_hbm.at[wid], ewv)
    plsc.subcore_barrier()

    def _chunk(i, carry):
        pltpu.sync_copy(ewv.at[i], acc_sh.at[rowv.at[i]], add=True)
        return carry

    lax.fori_loop(0, _CPT, _chunk, 0)
    plsc.subcore_barrier()
    pltpu.sync_copy(acc_sh.at[pl.ds(s * _RPT, _RPT)], zb)

    def _repl(g, carry):
        v = zb[pl.ds(g * 16, 16)]
        for j in range(16):
            rep[g * 16 + j] = jnp.full((16,), v[j], jnp.float32)
        return carry

    lax.fori_loop(0, _RPT // 16, _repl, 0)
    pltpu.sync_copy(rep, out_hbm.at[c, pl.ds(s * _RPT, _RPT)])


@functools.partial(
    pl.kernel,
    out_type=jax.ShapeDtypeStruct((2, _NP, 16), jnp.float32),
    mesh=_MESH,
    scratch_types=[
        pltpu.VMEM_SHARED((_NP, 16), jnp.float32),
        pltpu.VMEM((_CPT, _CH), jnp.int32),
        pltpu.VMEM((_CPT, _CH), jnp.int32),
        pltpu.VMEM((_EPT,), jnp.float32),
        pltpu.VMEM((4, _CH, 16), jnp.float32),
        pltpu.VMEM((4, _CH, 16), jnp.float32),
        pltpu.VMEM((_RPT, 16), jnp.float32),
        [pltpu.SemaphoreType.DMA] * 4,
        [pltpu.SemaphoreType.DMA] * 4,
    ],
    compiler_params=pltpu.CompilerParams(use_tc_tiling_on_sc=False),
)
def _sc_spmm(col3d_hbm, row3d_hbm, ew1d_hbm, y_hbm, out_hbm,
             acc_sh, colv, rowv, ewv, gb, sb, zb, gsem, ssem):
    c = lax.axis_index("c")
    s = lax.axis_index("s")
    wid = c * 16 + s

    def _zero(i, carry):
        zb[i] = jnp.zeros((16,), jnp.float32)
        return carry

    lax.fori_loop(0, _RPT, _zero, 0)
    pltpu.sync_copy(zb, acc_sh.at[pl.ds(s * _RPT, _RPT)])
    pltpu.sync_copy(col3d_hbm.at[wid], colv)
    pltpu.sync_copy(row3d_hbm.at[wid], rowv)
    pltpu.sync_copy(ew1d_hbm.at[pl.ds(wid * _EPT, _EPT)], ewv)
    plsc.subcore_barrier()

    def _fire_gather(ci, k):
        pltpu.async_copy(y_hbm.at[colv.at[ci]], gb.at[k], gsem[k])

    def _wait_gather(ci, k):
        pltpu.make_async_copy(y_hbm.at[colv.at[ci]], gb.at[k], gsem[k]).wait()

    def _fire_scatter(ci, k):
        pltpu.async_copy(sb.at[k], acc_sh.at[rowv.at[ci]], ssem[k], add=True)

    def _wait_scatter(ci, k):
        pltpu.make_async_copy(sb.at[k], acc_sh.at[rowv.at[ci]],
                              ssem[k]).wait()

    def _scale(ci, k):
        def _grp(g, carry):
            wv = ewv[pl.ds(ci * _CH + g * 16, 16)]
            e0 = g * 16
            for j in range(16):
                sb[k, e0 + j] = gb[k, e0 + j] * wv[j]
            return carry

        lax.fori_loop(0, _CH // 16, _grp, 0)

    for k in range(4):
        _fire_gather(k, k)
    for k in range(4):
        _wait_gather(k, k)
        _scale(k, k)
        _fire_gather(k + 4, k)
        _fire_scatter(k, k)

    def _body(g, carry):
        for k in range(4):
            ci = g * 4 + k
            _wait_gather(ci, k)
            _wait_scatter(ci - 4, k)
            _scale(ci, k)
            _fire_gather(ci + 4, k)
            _fire_scatter(ci, k)
        return carry

    lax.fori_loop(1, _CPT // 4 - 1, _body, 0)

    for k in range(4):
        ci = _CPT - 4 + k
        _wait_gather(ci, k)
        _wait_scatter(ci - 4, k)
        _scale(ci, k)
        _fire_scatter(ci, k)
    for k in range(4):
        _wait_scatter(_CPT - 4 + k, k)

    plsc.subcore_barrier()
    pltpu.sync_copy(acc_sh.at[pl.ds(s * _RPT, _RPT)],
                    out_hbm.at[c, pl.ds(s * _RPT, _RPT)])


def _tc1_body(xp_ref, w0_ref, w1_ref, degrep_ref,
              xw0_ref, y1p_ref, disrep_ref):
    d = degrep_ref[0] + degrep_ref[1]
    dis = jnp.where(d > 0, lax.rsqrt(jnp.maximum(d, 1e-30)), 0.0)
    disrep_ref[...] = dis
    xp = xp_ref[...]
    xw0_ref[...] = jnp.dot(xp, w0_ref[...],
                           preferred_element_type=jnp.float32)
    y1p_ref[...] = dis * jnp.dot(xp, w1_ref[...],
                                 preferred_element_type=jnp.float32)


def _tc2_body(xw0_ref, s1p_ref, disrep_ref, b1_ref, w0_ref, w1_ref,
              hw0_ref, y2p_ref):
    dis = disrep_ref[...]
    s1 = s1p_ref[0] + s1p_ref[1]
    h = jnp.maximum(xw0_ref[...] - dis * s1 + b1_ref[...], 0.0)
    hw0_ref[...] = jnp.dot(h, w0_ref[...],
                           preferred_element_type=jnp.float32)
    y2p_ref[...] = dis * jnp.dot(h, w1_ref[...],
                                 preferred_element_type=jnp.float32)


def _tc3_body(hw0_ref, s2p_ref, disrep_ref, b2_ref, ones_ref, out_ref):
    s2 = s2p_ref[0] + s2p_ref[1]
    o = hw0_ref[...] - disrep_ref[...] * s2 + b2_ref[...]
    m = jnp.max(o, axis=1, keepdims=True)
    z = o - m
    sums = jnp.dot(jnp.exp(z), ones_ref[...],
                   preferred_element_type=jnp.float32)
    out_ref[...] = z - jnp.log(sums)


_f32 = jnp.float32


def _block_diag8(w):
    k, m = w.shape
    out = jnp.zeros((8 * k, 8 * m), _f32)
    for i in range(8):
        out = out.at[i * k:(i + 1) * k, i * m:(i + 1) * m].set(w)
    return out


def kernel(x, edge_index, edge_weight, W1, b1, W2, b2):
    row = edge_index[0]
    col = edge_index[1]
    pad = _E_PAD - _E
    row3d = jnp.concatenate(
        [row, jnp.zeros((pad,), jnp.int32)]).reshape(32, _CPT, _CH)
    col3d = jnp.concatenate(
        [col, jnp.zeros((pad,), jnp.int32)]).reshape(32, _CPT, _CH)
    ew_p = jnp.concatenate([edge_weight, jnp.zeros((pad,), _f32)])
    ew3d = ew_p.reshape(32, _CPT, _CH)

    degrep = _sc_deg(row3d, ew3d).reshape(2, _PR, 128)

    w10bd = _block_diag8(W1[0])
    w11bd = _block_diag8(W1[1])
    w20bd = _block_diag8(W2[0])
    w21bd = _block_diag8(W2[1])
    onesbd = _block_diag8(jnp.ones((16, 16), _f32))
    b1rep = jnp.tile(b1, 8).reshape(1, 128)
    b2rep = jnp.tile(b2, 8).reshape(1, 128)

    xp = jnp.concatenate(
        [x, jnp.zeros((_NP - _N, 128), _f32)]).reshape(_PR, 1024)

    xw0, y1p, disrep = pl.pallas_call(
        _tc1_body,
        out_shape=[
            jax.ShapeDtypeStruct((_PR, 128), _f32),
            jax.ShapeDtypeStruct((_PR, 128), _f32),
            jax.ShapeDtypeStruct((_PR, 128), _f32),
        ],
    )(xp, w10bd, w11bd, degrep)

    s1p = _sc_spmm(col3d, row3d, ew_p, y1p.reshape(_NP, 16))
    s1p = s1p.reshape(2, _PR, 128)

    hw0, y2p = pl.pallas_call(
        _tc2_body,
        out_shape=[
            jax.ShapeDtypeStruct((_PR, 128), _f32),
            jax.ShapeDtypeStruct((_PR, 128), _f32),
        ],
    )(xw0, s1p, disrep, b1rep, w20bd, w21bd)

    s2p = _sc_spmm(col3d, row3d, ew_p, y2p.reshape(_NP, 16))
    s2p = s2p.reshape(2, _PR, 128)

    out = pl.pallas_call(
        _tc3_body,
        out_shape=jax.ShapeDtypeStruct((_PR, 128), _f32),
    )(hw0, s2p, disrep, b2rep, onesbd)
    return out.reshape(_NP, 16)[:_N]


# speedup vs baseline: 1.4001x; 1.1641x over previous
import functools

import jax
import jax.numpy as jnp
from jax import lax
from jax.experimental import pallas as pl
from jax.experimental.pallas import tpu as pltpu
from jax.experimental.pallas import tpu_sc as plsc

_N = 10000
_NP = 10240
_PR = _NP // 8
_E = 320000
_CH = 128
_CPT = 80
_EPT = _CH * _CPT
_E_PAD = 32 * _EPT
_RPT = _NP // 16

_MESH = plsc.VectorSubcoreMesh(core_axis_name="c", subcore_axis_name="s")


@functools.partial(
    pl.kernel,
    out_type=jax.ShapeDtypeStruct((2, _NP, 16), jnp.float32),
    mesh=_MESH,
    scratch_types=[
        pltpu.VMEM_SHARED((_NP,), jnp.float32),
        pltpu.VMEM((_CPT, _CH), jnp.int32),
        pltpu.VMEM((_CPT, _CH), jnp.float32),
        pltpu.VMEM((_RPT,), jnp.float32),
        pltpu.VMEM((_RPT, 16), jnp.float32),
    ],
    compiler_params=pltpu.CompilerParams(use_tc_tiling_on_sc=False),
)
def _sc_deg(row3d_hbm, ew3d_hbm, out_hbm, acc_sh, rowv, ewv, zb, rep):
    c = lax.axis_index("c")
    s = lax.axis_index("s")
    wid = c * 16 + s

    def _zero(i, carry):
        zb[pl.ds(i * 16, 16)] = jnp.zeros((16,), jnp.float32)
        return carry

    lax.fori_loop(0, _RPT // 16, _zero, 0)
    pltpu.sync_copy(zb, acc_sh.at[pl.ds(s * _RPT, _RPT)])
    pltpu.sync_copy(row3d_hbm.at[wid], rowv)
    pltpu.sync_copy(ew3d_hbm.at[wid], ewv)
    plsc.subcore_barrier()

    def _chunk(i, carry):
        pltpu.sync_copy(ewv.at[i], acc_sh.at[rowv.at[i]], add=True)
        return carry

    lax.fori_loop(0, _CPT, _chunk, 0)
    plsc.subcore_barrier()
    pltpu.sync_copy(acc_sh.at[pl.ds(s * _RPT, _RPT)], zb)

    def _repl(g, carry):
        v = zb[pl.ds(g * 16, 16)]
        for j in range(16):
            rep[g * 16 + j] = jnp.full((16,), v[j], jnp.float32)
        return carry

    lax.fori_loop(0, _RPT // 16, _repl, 0)
    pltpu.sync_copy(rep, out_hbm.at[c, pl.ds(s * _RPT, _RPT)])


@functools.partial(
    pl.kernel,
    out_type=jax.ShapeDtypeStruct((2, _NP, 16), jnp.float32),
    mesh=_MESH,
    scratch_types=[
        pltpu.VMEM_SHARED((_NP, 16), jnp.float32),
        pltpu.VMEM((_CPT, _CH), jnp.int32),
        pltpu.VMEM((_CPT, _CH), jnp.int32),
        pltpu.VMEM((_EPT,), jnp.float32),
        pltpu.VMEM((4, _CH, 16), jnp.float32),
        pltpu.VMEM((4, _CH, 16), jnp.float32),
        pltpu.VMEM((_RPT, 16), jnp.float32),
        [pltpu.SemaphoreType.DMA] * 4,
        [pltpu.SemaphoreType.DMA] * 4,
    ],
    compiler_params=pltpu.CompilerParams(use_tc_tiling_on_sc=False),
)
def _sc_spmm(col3d_hbm, row3d_hbm, ew1d_hbm, y_hbm, out_hbm,
             acc_sh, colv, rowv, ewv, gb, sb, zb, gsem, ssem):
    c = lax.axis_index("c")
    s = lax.axis_index("s")
    wid = c * 16 + s

    def _zero(i, carry):
        zb[i] = jnp.zeros((16,), jnp.float32)
        return carry

    lax.fori_loop(0, _RPT, _zero, 0)
    pltpu.sync_copy(zb, acc_sh.at[pl.ds(s * _RPT, _RPT)])
    pltpu.sync_copy(col3d_hbm.at[wid], colv)
    pltpu.sync_copy(row3d_hbm.at[wid], rowv)
    pltpu.sync_copy(ew1d_hbm.at[pl.ds(wid * _EPT, _EPT)], ewv)
    plsc.subcore_barrier()

    def _fire_gather(ci, k):
        pltpu.async_copy(y_hbm.at[colv.at[ci]], gb.at[k], gsem[k])

    def _wait_gather(ci, k):
        pltpu.make_async_copy(y_hbm.at[colv.at[ci]], gb.at[k], gsem[k]).wait()

    def _fire_scatter(ci, k):
        pltpu.async_copy(sb.at[k], acc_sh.at[rowv.at[ci]], ssem[k], add=True)

    def _wait_scatter(ci, k):
        pltpu.make_async_copy(sb.at[k], acc_sh.at[rowv.at[ci]],
                              ssem[k]).wait()

    def _scale(ci, k):
        def _grp(g, carry):
            wv = ewv[pl.ds(ci * _CH + g * 16, 16)]
            e0 = g * 16
            for j in range(16):
                sb[k, e0 + j] = gb[k, e0 + j] * wv[j]
            return carry

        lax.fori_loop(0, _CH // 16, _grp, 0)

    for k in range(4):
        _fire_gather(k, k)
    for k in range(4):
        _wait_gather(k, k)
        _scale(k, k)
        _fire_gather(k + 4, k)
        _fire_scatter(k, k)

    def _body(g, carry):
        for k in range(4):
            ci = g * 4 + k
            _wait_gather(ci, k)
            _wait_scatter(ci - 4, k)
            _scale(ci, k)
            _fire_gather(ci + 4, k)
            _fire_scatter(ci, k)
        return carry

    lax.fori_loop(1, _CPT // 4 - 1, _body, 0)

    for k in range(4):
        ci = _CPT - 4 + k
        _wait_gather(ci, k)
        _wait_scatter(ci - 4, k)
        _scale(ci, k)
        _fire_scatter(ci, k)
    for k in range(4):
        _wait_scatter(_CPT - 4 + k, k)

    plsc.subcore_barrier()
    pltpu.sync_copy(acc_sh.at[pl.ds(s * _RPT, _RPT)],
                    out_hbm.at[c, pl.ds(s * _RPT, _RPT)])


def _tc1_body(xp_ref, w0_ref, w1_ref, degrep_ref,
              xw0_ref, y1p_ref, disrep_ref):
    d = degrep_ref[0] + degrep_ref[1]
    dis = jnp.where(d > 0, lax.rsqrt(jnp.maximum(d, 1e-30)), 0.0)
    disrep_ref[...] = dis
    xp = xp_ref[...]
    xw0_ref[...] = jnp.dot(xp, w0_ref[...],
                           preferred_element_type=jnp.float32)
    y1p_ref[...] = dis * jnp.dot(xp, w1_ref[...],
                                 preferred_element_type=jnp.float32)


def _tc2_body(xw0_ref, s1p_ref, disrep_ref, b1_ref, w0_ref, w1_ref,
              hw0_ref, y2p_ref):
    dis = disrep_ref[...]
    s1 = s1p_ref[0] + s1p_ref[1]
    h = jnp.maximum(xw0_ref[...] - dis * s1 + b1_ref[...], 0.0)
    hw0_ref[...] = jnp.dot(h, w0_ref[...],
                           preferred_element_type=jnp.float32)
    y2p_ref[...] = dis * jnp.dot(h, w1_ref[...],
                                 preferred_element_type=jnp.float32)


def _tc3_body(hw0_ref, s2p_ref, disrep_ref, b2_ref, ones_ref, out_ref):
    s2 = s2p_ref[0] + s2p_ref[1]
    o = hw0_ref[...] - disrep_ref[...] * s2 + b2_ref[...]
    m = jnp.max(o, axis=1, keepdims=True)
    z = o - m
    sums = jnp.dot(jnp.exp(z), ones_ref[...],
                   preferred_element_type=jnp.float32)
    r = z - jnp.log(sums)
    out_ref[...] = r[:_N // 8, :]


_f32 = jnp.float32


def _block_diag8(w):
    k, m = w.shape
    eye = jnp.eye(8, dtype=_f32)
    return (eye[:, None, :, None] * w[None, :, None, :]).reshape(8 * k, 8 * m)


def kernel(x, edge_index, edge_weight, W1, b1, W2, b2):
    row = edge_index[0]
    col = edge_index[1]
    pad = _E_PAD - _E
    row3d = jnp.concatenate(
        [row, jnp.zeros((pad,), jnp.int32)]).reshape(32, _CPT, _CH)
    col3d = jnp.concatenate(
        [col, jnp.zeros((pad,), jnp.int32)]).reshape(32, _CPT, _CH)
    ew_p = jnp.concatenate([edge_weight, jnp.zeros((pad,), _f32)])
    ew3d = ew_p.reshape(32, _CPT, _CH)

    degrep = _sc_deg(row3d, ew3d).reshape(2, _PR, 128)

    w10bd = _block_diag8(W1[0])
    w11bd = _block_diag8(W1[1])
    w20bd = _block_diag8(W2[0])
    w21bd = _block_diag8(W2[1])
    onesbd = _block_diag8(jnp.ones((16, 16), _f32))
    b1rep = jnp.tile(b1, 8).reshape(1, 128)
    b2rep = jnp.tile(b2, 8).reshape(1, 128)

    xp = jnp.concatenate(
        [x, jnp.zeros((_NP - _N, 128), _f32)]).reshape(_PR, 1024)

    xw0, y1p, disrep = pl.pallas_call(
        _tc1_body,
        out_shape=[
            jax.ShapeDtypeStruct((_PR, 128), _f32),
            jax.ShapeDtypeStruct((_PR, 128), _f32),
            jax.ShapeDtypeStruct((_PR, 128), _f32),
        ],
    )(xp, w10bd, w11bd, degrep)

    s1p = _sc_spmm(col3d, row3d, ew_p, y1p.reshape(_NP, 16))
    s1p = s1p.reshape(2, _PR, 128)

    hw0, y2p = pl.pallas_call(
        _tc2_body,
        out_shape=[
            jax.ShapeDtypeStruct((_PR, 128), _f32),
            jax.ShapeDtypeStruct((_PR, 128), _f32),
        ],
    )(xw0, s1p, disrep, b1rep, w20bd, w21bd)

    s2p = _sc_spmm(col3d, row3d, ew_p, y2p.reshape(_NP, 16))
    s2p = s2p.reshape(2, _PR, 128)

    out = pl.pallas_call(
        _tc3_body,
        out_shape=jax.ShapeDtypeStruct((_N // 8, 128), _f32),
    )(hw0, s2p, disrep, b2rep, onesbd)
    return out.reshape(_N, 16)


# speedup vs baseline: 1.4811x; 1.0579x over previous
import functools

import jax
import jax.numpy as jnp
from jax import lax
from jax.experimental import pallas as pl
from jax.experimental.pallas import tpu as pltpu
from jax.experimental.pallas import tpu_sc as plsc

_N = 10000
_NP = 10240
_PR = _NP // 8
_E = 320000
_CH = 128
_CPT = 80
_EPT = _CH * _CPT
_E_PAD = 32 * _EPT
_NCHUNK = _E_PAD // _CH
_CPT0 = 104
_CPT1 = 56
_RPT = _NP // 16

_MESH = plsc.VectorSubcoreMesh(core_axis_name="c", subcore_axis_name="s")


@functools.partial(
    pl.kernel,
    out_type=jax.ShapeDtypeStruct((2, _NP, 16), jnp.float32),
    mesh=_MESH,
    scratch_types=[
        pltpu.VMEM_SHARED((_NP,), jnp.float32),
        pltpu.VMEM((_CPT, _CH), jnp.int32),
        pltpu.VMEM((_EPT,), jnp.float32),
        pltpu.VMEM((_RPT,), jnp.float32),
        pltpu.VMEM((_RPT, 16), jnp.float32),
    ],
    compiler_params=pltpu.CompilerParams(use_tc_tiling_on_sc=False),
)
def _sc_deg(row2d_hbm, ew1d_hbm, out_hbm, acc_sh, rowv, ewv, zb, rep):
    c = lax.axis_index("c")
    s = lax.axis_index("s")
    wid = c * 16 + s

    def _zero(i, carry):
        zb[pl.ds(i * 16, 16)] = jnp.zeros((16,), jnp.float32)
        return carry

    lax.fori_loop(0, _RPT // 16, _zero, 0)
    pltpu.sync_copy(zb, acc_sh.at[pl.ds(s * _RPT, _RPT)])
    pltpu.sync_copy(row2d_hbm.at[pl.ds(wid * _CPT, _CPT)], rowv)
    pltpu.sync_copy(ew1d_hbm.at[pl.ds(wid * _EPT, _EPT)], ewv)
    plsc.subcore_barrier()

    def _chunk(i, carry):
        pltpu.sync_copy(ewv.at[pl.ds(i * _CH, _CH)], acc_sh.at[rowv.at[i]],
                        add=True)
        return carry

    lax.fori_loop(0, _CPT, _chunk, 0)
    plsc.subcore_barrier()
    pltpu.sync_copy(acc_sh.at[pl.ds(s * _RPT, _RPT)], zb)

    def _repl(g, carry):
        v = zb[pl.ds(g * 16, 16)]
        for j in range(16):
            rep[g * 16 + j] = jnp.full((16,), v[j], jnp.float32)
        return carry

    lax.fori_loop(0, _RPT // 16, _repl, 0)
    pltpu.sync_copy(rep, out_hbm.at[c, pl.ds(s * _RPT, _RPT)])


@functools.partial(
    pl.kernel,
    out_type=jax.ShapeDtypeStruct((2, _NP, 16), jnp.float32),
    mesh=_MESH,
    scratch_types=[
        pltpu.VMEM_SHARED((_NP, 16), jnp.float32),
        pltpu.VMEM((_CPT0, _CH), jnp.int32),
        pltpu.VMEM((_CPT0, _CH), jnp.int32),
        pltpu.VMEM((_CPT0 * _CH,), jnp.float32),
        pltpu.VMEM((4, _CH, 16), jnp.float32),
        pltpu.VMEM((4, _CH, 16), jnp.float32),
        pltpu.VMEM((_RPT, 16), jnp.float32),
        [pltpu.SemaphoreType.DMA] * 4,
        [pltpu.SemaphoreType.DMA] * 4,
    ],
    compiler_params=pltpu.CompilerParams(use_tc_tiling_on_sc=False),
)
def _sc_spmm(col2d_hbm, row2d_hbm, ew1d_hbm, y_hbm, out_hbm,
             acc_sh, colv, rowv, ewv, gb, sb, zb, gsem, ssem):
    c = lax.axis_index("c")
    s = lax.axis_index("s")
    nch = jnp.where(c == 0, _CPT0, _CPT1)

    def _zero(i, carry):
        zb[i] = jnp.zeros((16,), jnp.float32)
        return carry

    lax.fori_loop(0, _RPT, _zero, 0)
    pltpu.sync_copy(zb, acc_sh.at[pl.ds(s * _RPT, _RPT)])

    @pl.when(c == 0)
    def _stage0():
        c0 = s * _CPT0
        pltpu.sync_copy(col2d_hbm.at[pl.ds(c0, _CPT0)], colv)
        pltpu.sync_copy(row2d_hbm.at[pl.ds(c0, _CPT0)], rowv)
        pltpu.sync_copy(ew1d_hbm.at[pl.ds(c0 * _CH, _CPT0 * _CH)], ewv)

    @pl.when(c == 1)
    def _stage1():
        c0 = 16 * _CPT0 + s * _CPT1
        pltpu.sync_copy(col2d_hbm.at[pl.ds(c0, _CPT1)],
                        colv.at[pl.ds(0, _CPT1)])
        pltpu.sync_copy(row2d_hbm.at[pl.ds(c0, _CPT1)],
                        rowv.at[pl.ds(0, _CPT1)])
        pltpu.sync_copy(ew1d_hbm.at[pl.ds(c0 * _CH, _CPT1 * _CH)],
                        ewv.at[pl.ds(0, _CPT1 * _CH)])

    plsc.subcore_barrier()

    def _fire_gather(ci, k):
        pltpu.async_copy(y_hbm.at[colv.at[ci]], gb.at[k], gsem[k])

    def _wait_gather(ci, k):
        pltpu.make_async_copy(y_hbm.at[colv.at[ci]], gb.at[k], gsem[k]).wait()

    def _fire_scatter(ci, k):
        pltpu.async_copy(sb.at[k], acc_sh.at[rowv.at[ci]], ssem[k], add=True)

    def _wait_scatter(ci, k):
        pltpu.make_async_copy(sb.at[k], acc_sh.at[rowv.at[ci]],
                              ssem[k]).wait()

    def _scale(ci, k):
        def _grp(g, carry):
            wv = ewv[pl.ds(ci * _CH + g * 16, 16)]
            e0 = g * 16
            for j in range(16):
                sb[k, e0 + j] = gb[k, e0 + j] * wv[j]
            return carry

        lax.fori_loop(0, _CH // 16, _grp, 0)

    for k in range(4):
        _fire_gather(k, k)
    for k in range(4):
        _wait_gather(k, k)
        _scale(k, k)
        _fire_gather(k + 4, k)
        _fire_scatter(k, k)

    def _body(g, carry):
        for k in range(4):
            ci = g * 4 + k
            _wait_gather(ci, k)
            _wait_scatter(ci - 4, k)
            _scale(ci, k)
            _fire_gather(ci + 4, k)
            _fire_scatter(ci, k)
        return carry

    lax.fori_loop(1, nch // 4 - 1, _body, 0)

    for k in range(4):
        ci = nch - 4 + k
        _wait_gather(ci, k)
        _wait_scatter(ci - 4, k)
        _scale(ci, k)
        _fire_scatter(ci, k)
    for k in range(4):
        _wait_scatter(nch - 4 + k, k)

    plsc.subcore_barrier()
    pltpu.sync_copy(acc_sh.at[pl.ds(s * _RPT, _RPT)],
                    out_hbm.at[c, pl.ds(s * _RPT, _RPT)])


def _tc1_body(xp_ref, w0_ref, w1_ref, degrep_ref,
              xw0_ref, y1p_ref, disrep_ref):
    d = degrep_ref[0] + degrep_ref[1]
    dis = jnp.where(d > 0, lax.rsqrt(jnp.maximum(d, 1e-30)), 0.0)
    disrep_ref[...] = dis
    xp = xp_ref[...]
    xw0_ref[...] = jnp.dot(xp, w0_ref[...],
                           preferred_element_type=jnp.float32)
    y1p_ref[...] = dis * jnp.dot(xp, w1_ref[...],
                                 preferred_element_type=jnp.float32)


def _tc2_body(xw0_ref, s1p_ref, disrep_ref, b1_ref, w0_ref, w1_ref,
              hw0_ref, y2p_ref):
    dis = disrep_ref[...]
    s1 = s1p_ref[0] + s1p_ref[1]
    h = jnp.maximum(xw0_ref[...] - dis * s1 + b1_ref[...], 0.0)
    hw0_ref[...] = jnp.dot(h, w0_ref[...],
                           preferred_element_type=jnp.float32)
    y2p_ref[...] = dis * jnp.dot(h, w1_ref[...],
                                 preferred_element_type=jnp.float32)


def _tc3_body(hw0_ref, s2p_ref, disrep_ref, b2_ref, ones_ref, out_ref):
    s2 = s2p_ref[0] + s2p_ref[1]
    o = hw0_ref[...] - disrep_ref[...] * s2 + b2_ref[...]
    m = jnp.max(o, axis=1, keepdims=True)
    z = o - m
    sums = jnp.dot(jnp.exp(z), ones_ref[...],
                   preferred_element_type=jnp.float32)
    r = z - jnp.log(sums)
    out_ref[...] = r[:_N // 8, :]


_f32 = jnp.float32


def _block_diag8(w):
    k, m = w.shape
    eye = jnp.eye(8, dtype=_f32)
    return (eye[:, None, :, None] * w[None, :, None, :]).reshape(8 * k, 8 * m)


def kernel(x, edge_index, edge_weight, W1, b1, W2, b2):
    row = edge_index[0]
    col = edge_index[1]
    pad = _E_PAD - _E
    row2d = jnp.concatenate(
        [row, jnp.zeros((pad,), jnp.int32)]).reshape(_NCHUNK, _CH)
    col2d = jnp.concatenate(
        [col, jnp.zeros((pad,), jnp.int32)]).reshape(_NCHUNK, _CH)
    ew_p = jnp.concatenate([edge_weight, jnp.zeros((pad,), _f32)])

    degrep = _sc_deg(row2d, ew_p).reshape(2, _PR, 128)

    w10bd = _block_diag8(W1[0])
    w11bd = _block_diag8(W1[1])
    w20bd = _block_diag8(W2[0])
    w21bd = _block_diag8(W2[1])
    onesbd = _block_diag8(jnp.ones((16, 16), _f32))
    b1rep = jnp.tile(b1, 8).reshape(1, 128)
    b2rep = jnp.tile(b2, 8).reshape(1, 128)

    xp = jnp.concatenate(
        [x, jnp.zeros((_NP - _N, 128), _f32)]).reshape(_PR, 1024)

    xw0, y1p, disrep = pl.pallas_call(
        _tc1_body,
        out_shape=[
            jax.ShapeDtypeStruct((_PR, 128), _f32),
            jax.ShapeDtypeStruct((_PR, 128), _f32),
            jax.ShapeDtypeStruct((_PR, 128), _f32),
        ],
    )(xp, w10bd, w11bd, degrep)

    s1p = _sc_spmm(col2d, row2d, ew_p, y1p.reshape(_NP, 16))
    s1p = s1p.reshape(2, _PR, 128)

    hw0, y2p = pl.pallas_call(
        _tc2_body,
        out_shape=[
            jax.ShapeDtypeStruct((_PR, 128), _f32),
            jax.ShapeDtypeStruct((_PR, 128), _f32),
        ],
    )(xw0, s1p, disrep, b1rep, w20bd, w21bd)

    s2p = _sc_spmm(col2d, row2d, ew_p, y2p.reshape(_NP, 16))
    s2p = s2p.reshape(2, _PR, 128)

    out = pl.pallas_call(
        _tc3_body,
        out_shape=jax.ShapeDtypeStruct((_N // 8, 128), _f32),
    )(hw0, s2p, disrep, b2rep, onesbd)
    return out.reshape(_N, 16)
